# baseline (speedup 1.0000x reference)
import functools

import jax
import jax.numpy as jnp
from jax import lax
from jax.experimental import pallas as pl
from jax.experimental.pallas import tpu as pltpu
from jax.experimental.pallas import tpu_sc as plsc

_info = plsc.get_sparse_core_info()
_NC, _NS = _info.num_cores, _info.num_subcores
_NW = _NC * _NS


@functools.lru_cache(maxsize=None)
def _make_gather(B: int, D: int):
    assert B % _NW == 0
    b_per_w = B // _NW
    chunk = 800
    while b_per_w % chunk:
        chunk //= 2
    nchunk = b_per_w // chunk
    mesh = plsc.VectorSubcoreMesh(core_axis_name="c", subcore_axis_name="s")

    @functools.partial(
        pl.kernel,
        mesh=mesh,
        compiler_params=pltpu.CompilerParams(use_tc_tiling_on_sc=False),
        out_type=jax.ShapeDtypeStruct((B, D), jnp.float32),
        scratch_types=[
            pltpu.VMEM((b_per_w,), jnp.int32),
            pltpu.VMEM((chunk, D), jnp.float32),
            pltpu.VMEM((chunk, D), jnp.float32),
            pltpu.SemaphoreType.DMA,
            pltpu.SemaphoreType.DMA,
            pltpu.SemaphoreType.DMA,
            pltpu.SemaphoreType.DMA,
        ],
    )
    def gather_kernel(table_hbm, idx_hbm, out_hbm, idx_v, buf0, buf1,
                      g0, g1, w0, w1):
        wid = lax.axis_index("s") * _NC + lax.axis_index("c")
        base = wid * b_per_w
        pltpu.sync_copy(idx_hbm.at[pl.ds(base, b_per_w)], idx_v)
        bufs = (buf0, buf1)
        gsems = (g0, g1)
        wsems = (w0, w1)
        gcp = [None, None]
        wcp = [None, None]
        gcp[0] = pltpu.async_copy(
            table_hbm.at[idx_v.at[pl.ds(0, chunk)]], buf0, g0)
        if nchunk > 1:
            gcp[1] = pltpu.async_copy(
                table_hbm.at[idx_v.at[pl.ds(chunk, chunk)]], buf1, g1)
        for c in range(nchunk):
            b = c % 2
            gcp[b].wait()
            wcp[b] = pltpu.async_copy(
                bufs[b], out_hbm.at[pl.ds(base + c * chunk, chunk)], wsems[b])
            if c + 2 < nchunk:
                wcp[b].wait()
                gcp[b] = pltpu.async_copy(
                    table_hbm.at[idx_v.at[pl.ds((c + 2) * chunk, chunk)]],
                    bufs[b], gsems[b])
        if nchunk > 1:
            wcp[(nchunk - 2) % 2].wait()
        wcp[(nchunk - 1) % 2].wait()

    return gather_kernel


def kernel(input, weight):
    B = input.shape[0] * input.shape[1]
    D = weight.shape[1]
    idx = input.reshape(B).astype(jnp.int32)
    out = _make_gather(B, D)(weight, idx)
    return out.reshape(input.shape[0], input.shape[1], D)


# speedup vs baseline: 1.0580x; 1.0580x over previous
import functools

import jax
import jax.numpy as jnp
from jax import lax
from jax.experimental import pallas as pl
from jax.experimental.pallas import tpu as pltpu
from jax.experimental.pallas import tpu_sc as plsc

_info = plsc.get_sparse_core_info()
_NC, _NS = _info.num_cores, _info.num_subcores
_NW = _NC * _NS


@functools.lru_cache(maxsize=None)
def _make_gather(B: int, Vp: int, D: int, DP: int):
    assert B % _NW == 0
    b_per_w = B // _NW
    chunk = 400
    while b_per_w % chunk:
        chunk //= 2
    nchunk = b_per_w // chunk
    mesh = plsc.VectorSubcoreMesh(core_axis_name="c", subcore_axis_name="s")

    @functools.partial(
        pl.kernel,
        mesh=mesh,
        compiler_params=pltpu.CompilerParams(use_tc_tiling_on_sc=False),
        out_type=jax.ShapeDtypeStruct((B, D), jnp.float32),
        scratch_types=[
            pltpu.VMEM((b_per_w,), jnp.int32),
            pltpu.VMEM((chunk, DP), jnp.float32),
            pltpu.VMEM((chunk, DP), jnp.float32),
            pltpu.SemaphoreType.DMA,
            pltpu.SemaphoreType.DMA,
            pltpu.SemaphoreType.DMA,
            pltpu.SemaphoreType.DMA,
        ],
    )
    def gather_kernel(table_hbm, idx_hbm, out_hbm, idx_v, buf0, buf1,
                      g0, g1, w0, w1):
        wid = lax.axis_index("s") * _NC + lax.axis_index("c")
        base = wid * b_per_w
        pltpu.sync_copy(idx_hbm.at[pl.ds(base, b_per_w)], idx_v)
        bufs = (buf0, buf1)
        gsems = (g0, g1)
        wsems = (w0, w1)
        gcp = [None, None]
        wcp = [None, None]
        gcp[0] = pltpu.async_copy(
            table_hbm.at[idx_v.at[pl.ds(0, chunk)]], buf0, g0)
        if nchunk > 1:
            gcp[1] = pltpu.async_copy(
                table_hbm.at[idx_v.at[pl.ds(chunk, chunk)]], buf1, g1)
        for c in range(nchunk):
            b = c % 2
            gcp[b].wait()
            wcp[b] = pltpu.async_copy(
                bufs[b].at[:, pl.ds(0, D)],
                out_hbm.at[pl.ds(base + c * chunk, chunk)], wsems[b])
            if c + 2 < nchunk:
                wcp[b].wait()
                gcp[b] = pltpu.async_copy(
                    table_hbm.at[idx_v.at[pl.ds((c + 2) * chunk, chunk)]],
                    bufs[b], gsems[b])
        if nchunk > 1:
            wcp[(nchunk - 2) % 2].wait()
        wcp[(nchunk - 1) % 2].wait()

    return gather_kernel


def kernel(input, weight):
    B = input.shape[0] * input.shape[1]
    V, D = weight.shape
    DP = 128
    Vp = (V + 7) // 8 * 8
    idx = input.reshape(B).astype(jnp.int32)
    wp = jnp.pad(weight, ((0, Vp - V), (0, DP - D)))
    out = _make_gather(B, Vp, D, DP)(wp, idx)
    return out.reshape(input.shape[0], input.shape[1], D)
